# Initial kernel scaffold; baseline (speedup 1.0000x reference)
import jax
import jax.numpy as jnp
from jax.experimental import pallas as pl


def kernel(z, xyz, cg_z, cg_xyz, mapping, nbr_list, cg_nbr_list, params):
    raise NotImplementedError("write your pallas kernel here")


# speedup vs baseline: 11.0392x; 11.0392x over previous
import functools

import jax
import jax.numpy as jnp
from jax import lax
from jax.experimental import pallas as pl
from jax.experimental.pallas import tpu as pltpu
from jax.experimental.pallas import tpu_sc as plsc

F = 128
NRBF = 20
CUT = 5.0
CGCUT = 12.5
NC, NS, L = 2, 16, 16
CH = 128


def _f32(x):
    return x.astype(jnp.float32)


def _sc_edge_diff(xyz_T, src, dst):
    E2 = src.shape[0]
    N = xyz_T.shape[1]
    nchunk = E2 // CH
    kmax = (nchunk + NC * NS - 1) // (NC * NS)
    mesh = plsc.VectorSubcoreMesh(core_axis_name="c", subcore_axis_name="s")

    @functools.partial(
        pl.kernel,
        out_type=jax.ShapeDtypeStruct((E2,), jnp.float32),
        mesh=mesh,
        compiler_params=pltpu.CompilerParams(needs_layout_passes=False),
        scratch_types=[
            pltpu.VMEM((CH,), jnp.int32),
            pltpu.VMEM((CH,), jnp.int32),
            pltpu.VMEM((N,), jnp.float32),
            pltpu.VMEM((N,), jnp.float32),
            pltpu.VMEM((N,), jnp.float32),
            pltpu.VMEM((CH,), jnp.float32),
        ],
    )
    def k(src_hbm, dst_hbm, x_hbm, y_hbm, z_hbm, out_hbm, sidx, didx,
          xv, yv, zv, obuf):
        cid = lax.axis_index("c")
        sid = lax.axis_index("s")
        wid = sid * NC + cid
        pltpu.sync_copy(x_hbm, xv)
        pltpu.sync_copy(y_hbm, yv)
        pltpu.sync_copy(z_hbm, zv)

        def chunk(kk, carry):
            g = kk * (NC * NS) + wid

            @pl.when(g < nchunk)
            def _():
                off = g * CH
                pltpu.sync_copy(src_hbm.at[pl.ds(off, CH)], sidx)
                pltpu.sync_copy(dst_hbm.at[pl.ds(off, CH)], didx)

                def grp(gi, carry2):
                    sv = sidx[pl.ds(gi * 16, 16)]
                    dv = didx[pl.ds(gi * 16, 16)]
                    acc = jnp.full((16,), 1e-08, jnp.float32)
                    for comp in (xv, yv, zv):
                        a = plsc.load_gather(comp, [sv])
                        b = plsc.load_gather(comp, [dv])
                        diff = a - b
                        acc = acc + diff * diff
                    obuf[pl.ds(gi * 16, 16)] = acc
                    return carry2

                lax.fori_loop(0, CH // 16, grp, None)
                pltpu.sync_copy(obuf, out_hbm.at[pl.ds(off, CH)])

            return carry

        lax.fori_loop(0, kmax, chunk, None)

    return k(src, dst, xyz_T[0], xyz_T[1], xyz_T[2])


def _sc_scatter(phi, w, src, seg, n_atoms):
    E2 = src.shape[0]
    Eh = E2 // 2
    assert w.shape[0] == Eh
    SCH = 80
    nchunk = E2 // SCH
    kmax = (nchunk + NC * NS - 1) // (NC * NS)
    accr = 10240
    rps = accr // NS
    mesh = plsc.VectorSubcoreMesh(core_axis_name="c", subcore_axis_name="s")

    npairs = (kmax + 1) // 2

    @functools.partial(
        pl.kernel,
        out_type=jax.ShapeDtypeStruct((NC, accr, F), jnp.float32),
        mesh=mesh,
        compiler_params=pltpu.CompilerParams(needs_layout_passes=False),
        scratch_types=[
            pltpu.VMEM((SCH,), jnp.int32),
            pltpu.VMEM((SCH,), jnp.int32),
            pltpu.VMEM((SCH,), jnp.int32),
            pltpu.VMEM((SCH,), jnp.int32),
            pltpu.VMEM((SCH,), jnp.int32),
            pltpu.VMEM((SCH,), jnp.int32),
            pltpu.VMEM((SCH, F), jnp.float32),
            pltpu.VMEM((SCH, F), jnp.float32),
            pltpu.VMEM((SCH, F), jnp.float32),
            pltpu.VMEM((SCH, F), jnp.float32),
            pltpu.VMEM_SHARED((accr, F), jnp.float32),
            pltpu.SemaphoreType.DMA,
            pltpu.SemaphoreType.DMA,
            pltpu.SemaphoreType.DMA,
            pltpu.SemaphoreType.DMA,
            pltpu.SemaphoreType.DMA,
            pltpu.SemaphoreType.DMA,
        ],
    )
    def k(phi_hbm, w_hbm, src_hbm, seg_hbm, out_hbm,
          sidx0, sidx1, didx0, didx1, didxS0, didxS1, wb0, wb1, rw0, rw1, acc,
          semi0, semi1, semd0, semd1, semw0, semw1):
        cid = lax.axis_index("c")
        sid = lax.axis_index("s")
        wid = sid * NC + cid
        stride = NC * NS
        sidx = (sidx0, sidx1)
        didx = (didx0, didx1)
        wb = (wb0, wb1)
        rw = (rw0, rw1)
        semi = (semi0, semi1)
        semd = (semd0, semd1)
        semw = (semw0, semw1)
        didxS = (didxS0, didxS1)

        def zrow(r, carry):
            for c in range(F // L):
                wb0[r, pl.ds(c * L, L)] = jnp.zeros((L,), jnp.float32)
            return carry

        lax.fori_loop(0, SCH, zrow, None)
        base = sid * rps
        for kk in range(rps // SCH):
            pltpu.sync_copy(wb0, acc.at[pl.ds(base + kk * SCH, SCH)])
        plsc.subcore_barrier()

        def goff(kk):
            return (kk * stride + wid) * SCH

        def issue_idx(kk, s):
            @pl.when(kk * stride + wid < nchunk)
            def _():
                off = goff(kk)
                pltpu.async_copy(seg_hbm.at[pl.ds(off, SCH)], didx[s], semi[s])
                pltpu.async_copy(src_hbm.at[pl.ds(off, SCH)], sidx[s], semi[s])

        def wait_check_issue_data(kk, s, pending):
            if pending is not None:
                @pl.when(pending)
                def _():
                    pltpu.make_async_copy(rw[s], acc.at[didxS[s]], semw[s]).wait()

            valid = kk * stride + wid < nchunk

            @pl.when(valid)
            def _():
                pltpu.make_async_copy(seg_hbm.at[pl.ds(0, SCH)], didx[s], semi[s]).wait()
                pltpu.make_async_copy(src_hbm.at[pl.ds(0, SCH)], sidx[s], semi[s]).wait()

            flag = valid & (jnp.min(didx[s][pl.ds(0, L)]) < n_atoms)

            @pl.when(flag)
            def _():
                off = goff(kk)
                woff = jnp.where(off >= Eh, off - Eh, off)
                pltpu.async_copy(phi_hbm.at[sidx[s]], rw[s], semd[s])
                pltpu.async_copy(w_hbm.at[pl.ds(woff, SCH)], wb[s], semd[s])

            return flag

        def process(flag, s):
            @pl.when(flag)
            def _():
                pltpu.make_async_copy(phi_hbm.at[sidx[s]], rw[s], semd[s]).wait()
                pltpu.make_async_copy(w_hbm.at[pl.ds(0, SCH)], wb[s], semd[s]).wait()

                def mrow(r, c2):
                    for c in range(F // L):
                        sl = pl.ds(c * L, L)
                        rw[s][r, sl] = rw[s][r, sl] * wb[s][r, sl]
                    return c2

                lax.fori_loop(0, SCH, mrow, None)
                for j in range(SCH // L):
                    sl = pl.ds(j * L, L)
                    didxS[s][sl] = didx[s][sl]
                pltpu.async_copy(rw[s], acc.at[didxS[s]], semw[s], add=True)

        issue_idx(0, 0)
        flag0 = wait_check_issue_data(0, 0, None)
        issue_idx(1, 1)

        def pair(k2, carry):
            flag_a, flag_b_prev = carry
            ka = 2 * k2
            process(flag_a, 0)
            flag_b = wait_check_issue_data(ka + 1, 1, flag_b_prev)
            issue_idx(ka + 2, 0)
            process(flag_b, 1)
            flag_a2 = wait_check_issue_data(ka + 2, 0, flag_a)
            issue_idx(ka + 3, 1)
            return (flag_a2, flag_b)

        fa, fb = lax.fori_loop(0, npairs, pair, (flag0, flag0 & False))

        @pl.when(fa)
        def _():
            pltpu.make_async_copy(rw[0], acc.at[didxS[0]], semw[0]).wait()

        @pl.when(fb)
        def _():
            pltpu.make_async_copy(rw[1], acc.at[didxS[1]], semw[1]).wait()

        plsc.subcore_barrier()
        pltpu.sync_copy(acc.at[pl.ds(base, rps)], out_hbm.at[cid, pl.ds(base, rps)])

    return k(phi, w, src, seg)


def _iota(n, dtype=jnp.int32):
    return lax.broadcasted_iota(dtype, (1, n), 1)


def _dist_rbf_w(r16, Wdm, bdm, cutoff):
    r3 = r16[:, 0:3]
    d = jnp.sqrt(jnp.sum(r3 * r3, axis=1, keepdims=True) + 1e-08)
    n = _f32(_iota(NRBF)) + 1.0
    rbf = jnp.sin(n * (jnp.pi / cutoff) * d) / d
    u = jnp.dot(rbf, Wdm, preferred_element_type=jnp.float32) + bdm
    env = jnp.where(d < cutoff, 0.5 * (jnp.cos((jnp.pi / cutoff) * d) + 1.0), 0.0)
    return u * env


def _mlp_mid(x, W1, b1, W2m, b2m):
    u = jnp.dot(x, W1, preferred_element_type=jnp.float32) + b1
    u = u * jax.nn.sigmoid(u)
    return jnp.dot(u, W2m, preferred_element_type=jnp.float32) + b2m


def _tc_wmat(d23, WdE, cutoff):
    R = d23.shape[0]
    EB = d23.shape[2]
    E2 = R * EB

    def body(d2_ref, w_ref, out_ref):
        dd = d2_ref[0, :, :]
        d = jnp.sqrt(dd)
        theta = (jnp.pi / cutoff) * d
        row = lax.broadcasted_iota(jnp.int32, (24, 1), 0)
        nn = jnp.where(row < NRBF, _f32(row) + 1.0,
                       jnp.where(row == NRBF, 1.0, 0.0))
        off = jnp.where(row == NRBF, jnp.float32(jnp.pi / 2), 0.0)
        sins = jnp.sin(nn * theta + off)
        env = jnp.where(d < cutoff, 0.5 * (sins[NRBF:NRBF + 1, :] + 1.0), 0.0)
        renv = jnp.where(row < NRBF, sins * (env / d),
                         jnp.where(row == NRBF, env, 0.0))
        out_ref[:, :] = lax.dot_general(
            renv, w_ref[:, :], (((0,), (0,)), ((), ())),
            preferred_element_type=jnp.float32)

    return pl.pallas_call(
        body,
        grid=(R,),
        in_specs=[
            pl.BlockSpec((1, 1, EB), lambda i: (i, 0, 0)),
            pl.BlockSpec((24, F), lambda i: (0, 0)),
        ],
        out_specs=pl.BlockSpec((EB, F), lambda i: (i, 0)),
        out_shape=jax.ShapeDtypeStruct((E2, F), jnp.float32),
    )(d23, WdE)


def _tc_atom_prep(z3, map3, cgz_row, aE, rE, cg_xyz_pad, xyz_pad,
                  Wcdm_all, bcdm_all):
    N = xyz_pad.shape[0]
    AB = 400
    grid = (N // AB,)
    nz = aE.shape[0]
    nr = rE.shape[0]
    ncg = cg_xyz_pad.shape[0]

    def body(z_ref, m_ref, cgz_ref, aE_ref, rE_ref, cgx_ref, xyz_ref,
             wd_ref, bd_ref, h0_ref, wc_ref):
        zb = z_ref[0, 0, :].reshape(AB, 1)
        mb = m_ref[0, 0, :].reshape(AB, 1)
        ohz = _f32(zb == _iota(nz))
        ohm = _f32(mb == _iota(ncg))
        cgzm = jnp.sum(ohm * cgz_ref[:, :], axis=1, keepdims=True)
        ohcg = _f32(cgzm == _f32(_iota(nr)))
        h0_ref[:, :] = (jnp.dot(ohz, aE_ref[:, :], preferred_element_type=jnp.float32)
                        + jnp.dot(ohcg, rE_ref[:, :], preferred_element_type=jnp.float32))
        cgx = jnp.dot(ohm, cgx_ref[:, :], preferred_element_type=jnp.float32)
        r16 = xyz_ref[:, :] - cgx
        wc_ref[:, :] = _dist_rbf_w(r16, wd_ref[:, :], bd_ref[:, :], CGCUT)

    return pl.pallas_call(
        body,
        grid=grid,
        in_specs=[
            pl.BlockSpec((1, 1, AB), lambda i: (i, 0, 0)),
            pl.BlockSpec((1, 1, AB), lambda i: (i, 0, 0)),
            pl.BlockSpec((1, ncg), lambda i: (0, 0)),
            pl.BlockSpec((nz, 2 * F), lambda i: (0, 0)),
            pl.BlockSpec((nr, 2 * F), lambda i: (0, 0)),
            pl.BlockSpec((ncg, 16), lambda i: (0, 0)),
            pl.BlockSpec((AB, 16), lambda i: (i, 0)),
            pl.BlockSpec((NRBF, 3 * F), lambda i: (0, 0)),
            pl.BlockSpec((1, 3 * F), lambda i: (0, 0)),
        ],
        out_specs=[
            pl.BlockSpec((AB, 2 * F), lambda i: (i, 0)),
            pl.BlockSpec((AB, 3 * F), lambda i: (i, 0)),
        ],
        out_shape=[
            jax.ShapeDtypeStruct((N, 2 * F), jnp.float32),
            jax.ShapeDtypeStruct((N, 3 * F), jnp.float32),
        ],
    )(z3, map3, cgz_row, aE, rE, cg_xyz_pad, xyz_pad, Wcdm_all, bcdm_all)


def _tc_phi(h, W1, b1, W2m, b2m):
    N = h.shape[0]
    AB = 400
    grid = (N // AB,)

    def body(h_ref, w1_ref, b1_ref, w2_ref, b2_ref, out_ref):
        out_ref[:, :] = _mlp_mid(h_ref[:, :], w1_ref[:, :], b1_ref[:, :],
                                 w2_ref[:, :], b2_ref[:, :])

    return pl.pallas_call(
        body,
        grid=grid,
        in_specs=[
            pl.BlockSpec((AB, F), lambda i: (i, 0)),
            pl.BlockSpec((F, F), lambda i: (0, 0)),
            pl.BlockSpec((1, F), lambda i: (0, 0)),
            pl.BlockSpec((F, F), lambda i: (0, 0)),
            pl.BlockSpec((1, F), lambda i: (0, 0)),
        ],
        out_specs=pl.BlockSpec((AB, F), lambda i: (i, 0)),
        out_shape=jax.ShapeDtypeStruct((N, F), jnp.float32),
    )(h, W1, b1, W2m, b2m)


def _tc_update(h_prev, ds_part, Aacc, wc_all, t, cgW, msgW, map3):
    N = h_prev.shape[0]
    AB = 400
    grid = (N // AB,)
    nblk = N // AB
    last = (t == 3)
    first = (t == 1)
    accr = ds_part.shape[1]
    ncg = 1000

    def body(*refs):
        refs = list(refs)
        h_ref = refs.pop(0)
        ds_ref = refs.pop(0)
        a_in = None if first else refs.pop(0)
        wc_ref = refs.pop(0)
        cw1, cb1, cw2, cb2 = refs.pop(0), refs.pop(0), refs.pop(0), refs.pop(0)
        if last:
            m_ref = refs.pop(0)
            h_out, H_out, hsum, csum = refs
        else:
            mw1, mb1, mw2, mb2 = refs.pop(0), refs.pop(0), refs.pop(0), refs.pop(0)
            h_out, a_out, phi_out = refs

        hn = h_ref[:, :] + ds_ref[0] + ds_ref[1]
        h_out[:, :] = hn
        contrib = (_mlp_mid(hn, cw1[:, :], cb1[:, :], cw2[:, :], cb2[:, :])
                   * wc_ref[:, (t - 1) * F:t * F])
        if first:
            a_new = hn + contrib
        else:
            a_new = a_in[:, :] + contrib
        if last:
            i = pl.program_id(0)
            mb = m_ref[0, 0, :].reshape(AB, 1)
            ohm = _f32(mb == _iota(ncg))
            part = lax.dot_general(ohm, a_new, (((0,), (0,)), ((), ())),
                                   preferred_element_type=jnp.float32)
            cpart = lax.dot_general(ohm, jnp.ones((AB, 8), jnp.float32),
                                    (((0,), (0,)), ((), ())),
                                    preferred_element_type=jnp.float32)

            @pl.when(i == 0)
            def _():
                hsum[:, :] = jnp.zeros((ncg, F), jnp.float32)
                csum[:, :] = jnp.zeros((ncg, 8), jnp.float32)

            hsum[:, :] = hsum[:, :] + part
            csum[:, :] = csum[:, :] + cpart

            @pl.when(i == nblk - 1)
            def _():
                cnt = jnp.clip(csum[:, 0:1], 1.0, None)
                H_out[:, :] = hsum[:, :] / cnt
        else:
            a_out[:, :] = a_new
            phi_out[:, :] = _mlp_mid(hn, mw1[:, :], mb1[:, :], mw2[:, :], mb2[:, :])

    in_specs = [
        pl.BlockSpec((AB, F), lambda i: (i, 0)),
        pl.BlockSpec((NC, AB, F), lambda i: (0, i, 0)),
    ]
    operands = [h_prev, ds_part]
    if not first:
        in_specs.append(pl.BlockSpec((AB, F), lambda i: (i, 0)))
        operands.append(Aacc)
    in_specs.append(pl.BlockSpec((AB, 3 * F), lambda i: (i, 0)))
    operands.append(wc_all)
    wspec = [
        pl.BlockSpec((F, F), lambda i: (0, 0)),
        pl.BlockSpec((1, F), lambda i: (0, 0)),
        pl.BlockSpec((F, F), lambda i: (0, 0)),
        pl.BlockSpec((1, F), lambda i: (0, 0)),
    ]
    in_specs += wspec
    operands += list(cgW)
    scratch_shapes = []
    if last:
        in_specs.append(pl.BlockSpec((1, 1, AB), lambda i: (i, 0, 0)))
        operands.append(map3)
        out_specs = [
            pl.BlockSpec((AB, F), lambda i: (i, 0)),
            pl.BlockSpec((ncg, F), lambda i: (0, 0)),
        ]
        out_shape = [
            jax.ShapeDtypeStruct((N, F), jnp.float32),
            jax.ShapeDtypeStruct((ncg, F), jnp.float32),
        ]
        scratch_shapes = [
            pltpu.VMEM((ncg, F), jnp.float32),
            pltpu.VMEM((ncg, 8), jnp.float32),
        ]
    else:
        in_specs += wspec
        operands += list(msgW)
        out_specs = [pl.BlockSpec((AB, F), lambda i: (i, 0))] * 3
        out_shape = [jax.ShapeDtypeStruct((N, F), jnp.float32)] * 3

    return pl.pallas_call(
        body,
        grid=grid,
        in_specs=in_specs,
        out_specs=out_specs,
        out_shape=out_shape,
        scratch_shapes=scratch_shapes,
    )(*operands)


def kernel(z, xyz, cg_z, cg_xyz, mapping, nbr_list, cg_nbr_list, params):
    N = xyz.shape[0]
    ncg = cg_xyz.shape[0]
    E = nbr_list.shape[0]

    nbr = nbr_list.astype(jnp.int32)
    b0 = nbr[:, 0]
    b1 = nbr[:, 1]
    directed = (b0 > b1).any() & (b1 > b0).any()
    src = jnp.concatenate([b1, b0])
    seg = jnp.concatenate([b0, jnp.where(directed, N, b1)])

    xyz_T = _f32(xyz).T
    xyz_pad = jnp.zeros((N, 16), jnp.float32).at[:, :3].set(_f32(xyz))
    cg_xyz_pad = jnp.zeros((ncg, 16), jnp.float32).at[:, :3].set(_f32(cg_xyz))
    z3 = z.astype(jnp.int32).reshape(25, 1, N // 25)
    map3 = mapping.astype(jnp.int32).reshape(25, 1, N // 25)
    cgz_row = _f32(cg_z).reshape(1, ncg)
    aE = jnp.zeros((56, 2 * F), jnp.float32).at[:50, :64].set(params['atom_embed'])
    rE = jnp.zeros((32, 2 * F), jnp.float32).at[:25, 64:128].set(params['res_embed'])

    def mid(p):
        return (p['W1'], p['b1'].reshape(1, F),
                p['W2'][:, F:2 * F], p['b2'][F:2 * F].reshape(1, F))

    def wd_mid(ps):
        Wdm = jnp.concatenate([p['Wd'][:, F:2 * F] for p in ps], axis=1)
        bdm = jnp.concatenate([p['bd'][F:2 * F] for p in ps]).reshape(1, 3 * F)
        return Wdm, bdm

    def wde(p):
        return (jnp.zeros((24, F), jnp.float32)
                .at[:NRBF].set(p['Wd'][:, F:2 * F])
                .at[NRBF].set(p['bd'][F:2 * F]))

    Wcdm_all, bcdm_all = wd_mid(params['cg'])

    d2 = _sc_edge_diff(xyz_T, b1, b0)
    d23 = d2.reshape(125, 1, E // 125)
    h0, wc_all = _tc_atom_prep(z3, map3, cgz_row, aE, rE, cg_xyz_pad, xyz_pad,
                               Wcdm_all, bcdm_all)
    w0 = _tc_wmat(d23, wde(params['msg'][0]), CUT)
    phi = _tc_phi(h0, *mid(params['msg'][0]))
    ds0 = _sc_scatter(phi, w0, src, seg, N)
    w1 = _tc_wmat(d23, wde(params['msg'][1]), CUT)
    h1, Aacc, phi = _tc_update(h0, ds0, None, wc_all, 1,
                               mid(params['cg'][0]), mid(params['msg'][1]), None)
    ds1 = _sc_scatter(phi, w1, src, seg, N)
    w2 = _tc_wmat(d23, wde(params['msg'][2]), CUT)
    h2, Aacc, phi = _tc_update(h1, ds1, Aacc, wc_all, 2,
                               mid(params['cg'][1]), mid(params['msg'][2]), None)
    ds2 = _sc_scatter(phi, w2, src, seg, N)
    h3, H = _tc_update(h2, ds2, Aacc, wc_all, 3,
                       mid(params['cg'][2]), None, map3)
    return (H, h3)
